# Initial kernel scaffold; baseline (speedup 1.0000x reference)
import jax
import jax.numpy as jnp
from jax.experimental import pallas as pl


def kernel(input_ids, emb_table, W1, b1, W2, b2):
    raise NotImplementedError("write your pallas kernel here")


# speedup vs baseline: 2.8112x; 2.8112x over previous
import functools

import jax
import jax.numpy as jnp
from jax import lax
from jax.experimental import pallas as pl
from jax.experimental.pallas import tpu as pltpu
from jax.experimental.pallas import tpu_sc as plsc

_INFO = plsc.get_sparse_core_info()
_NC, _NS, _L = _INFO.num_cores, _INFO.num_subcores, _INFO.num_lanes
_NW = _NC * _NS

_BATCH = 4096
_SEQ = 50
_D = 768
_H = 256
_NE = 28
_HV = _H // 16

_B_PER_W = _BATCH // _NW
_ROWS_PER_CHUNK = 2
_IDS_REAL = _ROWS_PER_CHUNK * _SEQ
_IDS_PAD = 104
_CHUNKS = _B_PER_W // _ROWS_PER_CHUNK

_BM = 512
_MB = -(-30522 // _BM)


def _t1_body(e_ref, w_ref, o_ref):
    o_ref[...] = jnp.dot(e_ref[...], w_ref[...],
                         preferred_element_type=jnp.float32)


def _pool_body(ids_hbm, t1_hbm, out_hbm, idx_v, rows_v, acc_v, sem):
    wid = lax.axis_index("s") * _NC + lax.axis_index("c")
    pltpu.sync_copy(ids_hbm.at[wid], idx_v)

    def chunk_body(c, carry):
        pltpu.async_copy(t1_hbm.at[idx_v.at[c]], rows_v, sem).wait()
        for k in range(_ROWS_PER_CHUNK):
            def seq_body(s, acc):
                r = k * _SEQ + s
                return tuple(acc[j] + rows_v[r, pl.ds(16 * j, 16)]
                             for j in range(_HV))
            acc = lax.fori_loop(
                0, _SEQ, seq_body,
                tuple(jnp.zeros((16,), jnp.float32) for _ in range(_HV)))
            row = c * _ROWS_PER_CHUNK + k
            for j in range(_HV):
                acc_v[row, pl.ds(16 * j, 16)] = acc[j]
        return carry

    lax.fori_loop(0, _CHUNKS, chunk_body, 0)
    pltpu.sync_copy(acc_v, out_hbm.at[pl.ds(wid * _B_PER_W, _B_PER_W)])


def _mlp_body(p_ref, b1_ref, w2_ref, b2_ref, o_ref):
    h = jnp.maximum(p_ref[...] * (1.0 / _SEQ) + b1_ref[...], 0.0)
    o_ref[...] = jnp.dot(h, w2_ref[...],
                         preferred_element_type=jnp.float32) + b2_ref[...]


def kernel(input_ids, emb_table, W1, b1, W2, b2):
    T1 = pl.pallas_call(
        _t1_body,
        grid=(_MB,),
        in_specs=[pl.BlockSpec((_BM, _D), lambda i: (i, 0)),
                  pl.BlockSpec((_D, _H), lambda i: (0, 0))],
        out_specs=pl.BlockSpec((_BM, _H), lambda i: (i, 0)),
        out_shape=jax.ShapeDtypeStruct((_MB * _BM, _H), jnp.float32),
    )(emb_table, W1)

    ids = input_ids.astype(jnp.int32).reshape(_NW, _CHUNKS, _IDS_REAL)
    ids = jnp.pad(ids, ((0, 0), (0, 0), (0, _IDS_PAD - _IDS_REAL)))

    pool = functools.partial(
        pl.kernel,
        mesh=plsc.VectorSubcoreMesh(core_axis_name="c", subcore_axis_name="s"),
        out_type=jax.ShapeDtypeStruct((_BATCH, _H), jnp.float32),
        scratch_types=[
            pltpu.VMEM((_CHUNKS, _IDS_PAD), jnp.int32),
            pltpu.VMEM((_IDS_PAD, _H), jnp.float32),
            pltpu.VMEM((_B_PER_W, _H), jnp.float32),
            pltpu.SemaphoreType.DMA,
        ],
    )(_pool_body)
    P = pool(ids, T1)

    return pl.pallas_call(
        _mlp_body,
        grid=(8,),
        in_specs=[pl.BlockSpec((_BATCH // 8, _H), lambda i: (i, 0)),
                  pl.BlockSpec((1, _H), lambda i: (0, 0)),
                  pl.BlockSpec((_H, _NE), lambda i: (0, 0)),
                  pl.BlockSpec((1, _NE), lambda i: (0, 0))],
        out_specs=pl.BlockSpec((_BATCH // 8, _NE), lambda i: (i, 0)),
        out_shape=jax.ShapeDtypeStruct((_BATCH, _NE), jnp.float32),
    )(P, b1[None], W2, b2[None])
